# baseline (speedup 1.0000x reference)
import jax
import jax.numpy as jnp
from jax import lax
from jax.experimental import pallas as pl
from jax.experimental.pallas import tpu as pltpu
from jax.experimental.pallas import tpu_sc as plsc

N = 10000
E = 320000
D = 128
H = 4
DH = D // H

NC = 2
NS = 16
EPT = E // NS
B = 32
NBLK = EPT // B
N2 = 10240
RPT = N2 // NS


def _sc_edge_body(q_hbm, kv_hbm, rows_hbm, cols_hbm, z64_hbm, z16_hbm,
                  outs_hbm, outn_hbm,
                  rows_all, cols_all,
                  rows_va, rows2_va, cols2_va, qr_va, kvc_va, outs_va, outn_va,
                  rows_vb, rows2_vb, cols2_vb, qr_vb, kvc_vb, outs_vb, outn_vb,
                  acc_s, acc_n,
                  semqa, semka, semsa, semna, semqb, semkb, semsb, semnb):
    c = lax.axis_index("c")
    s = lax.axis_index("s")

    r0 = s * RPT
    pltpu.sync_copy(z64_hbm, outs_va)
    pltpu.sync_copy(z16_hbm, outn_va)
    pltpu.sync_copy(z16_hbm, outn_vb)
    for i in range(RPT // B):
        pltpu.sync_copy(outs_va, acc_s.at[pl.ds(r0 + i * B, B)])
        pltpu.sync_copy(outn_va, acc_n.at[pl.ds(r0 + i * B, B)])

    pltpu.sync_copy(rows_hbm.at[pl.ds(s * EPT, EPT)], rows_all)
    pltpu.sync_copy(cols_hbm.at[pl.ds(s * EPT, EPT)], cols_all)

    lanes = lax.iota(jnp.int32, 16)
    rowpat = lanes // 2
    colpat = lanes % 2
    coff = jnp.full((16,), c * N, jnp.int32)
    zv = jnp.zeros((16,), jnp.float32)

    plsc.subcore_barrier()

    def prep(bb, rows_v, rows2_v, cols2_v):
        for t in range(B // 16):
            rv = rows_all[pl.ds(bb * B + t * 16, 16)]
            cv = cols_all[pl.ds(bb * B + t * 16, 16)]
            rows_v[pl.ds(t * 16, 16)] = rv
            rows2_v[pl.ds(t * 16, 16)] = rv + coff
            cols2_v[pl.ds(t * 16, 16)] = cv + coff

    def fire(rows2_v, cols2_v, qr_v, kvc_v, semq, semk):
        pltpu.async_copy(q_hbm.at[rows2_v], qr_v, semq)
        pltpu.async_copy(kv_hbm.at[cols2_v], kvc_v, semk)

    def wait_gather(rows2_v, cols2_v, qr_v, kvc_v, semq, semk):
        pltpu.make_async_copy(q_hbm.at[rows2_v], qr_v, semq).wait()
        pltpu.make_async_copy(kv_hbm.at[cols2_v], kvc_v, semk).wait()

    def compute(qr_v, kvc_v, outs_v, outn_v):
        for g in range(B // 8):
            parts = []
            for j in range(16):
                e = 8 * g + j // 2
                hl = j % 2
                q0 = qr_v[e, pl.ds(hl * DH, 16)]
                q1 = qr_v[e, pl.ds(hl * DH + 16, 16)]
                k0 = kvc_v[e, pl.ds(hl * DH, 16)]
                k1 = kvc_v[e, pl.ds(hl * DH + 16, 16)]
                sv = jnp.full((16,), jnp.sum(q0 * k0 + q1 * k1))
                parts.append(jnp.where(lanes == j, sv, zv))
            while len(parts) > 1:
                parts = [parts[i] + parts[i + 1] for i in range(0, len(parts), 2)]
            attvec = jnp.minimum(jnp.maximum(parts[0], -10.0), 10.0)
            ea = jnp.exp(attvec)
            plsc.store_scatter(outn_v, [rowpat + 8 * g, colpat], ea)
            for j in range(16):
                e = 8 * g + j // 2
                hl = j % 2
                av = jnp.full((16,), ea[j])
                v0 = kvc_v[e, pl.ds(64 + hl * DH, 16)]
                v1 = kvc_v[e, pl.ds(64 + hl * DH + 16, 16)]
                outs_v[e, pl.ds(hl * DH, 16)] = av * v0
                outs_v[e, pl.ds(hl * DH + 16, 16)] = av * v1

    def fire_scatter(outs_v, outn_v, rows_v, sems, semn):
        pltpu.async_copy(outs_v, acc_s.at[rows_v], sems, add=True)
        pltpu.async_copy(outn_v, acc_n.at[rows_v], semn, add=True)

    def wait_scatter(outs_v, outn_v, rows_v, sems, semn):
        pltpu.make_async_copy(outs_v, acc_s.at[rows_v], sems).wait()
        pltpu.make_async_copy(outn_v, acc_n.at[rows_v], semn).wait()

    prep(0, rows_va, rows2_va, cols2_va)
    fire(rows2_va, cols2_va, qr_va, kvc_va, semqa, semka)

    NPAIR = (NBLK - 1) // 2

    def pair(k, carry):
        bA = 2 * k
        @pl.when(k > 0)
        def _():
            wait_scatter(outs_vb, outn_vb, rows_vb, semsb, semnb)
        prep(bA + 1, rows_vb, rows2_vb, cols2_vb)
        fire(rows2_vb, cols2_vb, qr_vb, kvc_vb, semqb, semkb)
        wait_gather(rows2_va, cols2_va, qr_va, kvc_va, semqa, semka)
        compute(qr_va, kvc_va, outs_va, outn_va)
        fire_scatter(outs_va, outn_va, rows_va, semsa, semna)
        @pl.when(k < NPAIR - 1)
        def _():
            wait_scatter(outs_va, outn_va, rows_va, semsa, semna)
            prep(bA + 2, rows_va, rows2_va, cols2_va)
            fire(rows2_va, cols2_va, qr_va, kvc_va, semqa, semka)
        wait_gather(rows2_vb, cols2_vb, qr_vb, kvc_vb, semqb, semkb)
        compute(qr_vb, kvc_vb, outs_vb, outn_vb)
        fire_scatter(outs_vb, outn_vb, rows_vb, semsb, semnb)
        return carry

    lax.fori_loop(0, NPAIR, pair, 0)

    wait_scatter(outs_va, outn_va, rows_va, semsa, semna)
    wait_scatter(outs_vb, outn_vb, rows_vb, semsb, semnb)
    prep(NBLK - 1, rows_va, rows2_va, cols2_va)
    fire(rows2_va, cols2_va, qr_va, kvc_va, semqa, semka)
    wait_gather(rows2_va, cols2_va, qr_va, kvc_va, semqa, semka)
    compute(qr_va, kvc_va, outs_va, outn_va)
    fire_scatter(outs_va, outn_va, rows_va, semsa, semna)
    wait_scatter(outs_va, outn_va, rows_va, semsa, semna)

    plsc.subcore_barrier()

    for i in range(RPT // B):
        pltpu.sync_copy(acc_s.at[pl.ds(r0 + i * B, B)], outs_va)
        pltpu.sync_copy(outs_va, outs_hbm.at[c, pl.ds(r0 + i * B, B)])
        pltpu.sync_copy(acc_n.at[pl.ds(r0 + i * B, B)], outn_va)
        pltpu.sync_copy(outn_va, outn_hbm.at[c, pl.ds(r0 + i * B, B)])


_sc_edge = pl.kernel(
    _sc_edge_body,
    out_type=(
        jax.ShapeDtypeStruct((NC, N2, 64), jnp.float32),
        jax.ShapeDtypeStruct((NC, N2, 16), jnp.float32),
    ),
    mesh=plsc.VectorSubcoreMesh(core_axis_name="c", subcore_axis_name="s"),
    compiler_params=pltpu.CompilerParams(
        needs_layout_passes=False, use_tc_tiling_on_sc=False),
    scratch_types=(
        [
            pltpu.VMEM((EPT,), jnp.int32),
            pltpu.VMEM((EPT,), jnp.int32),
        ]
        + 2 * [
            pltpu.VMEM((B,), jnp.int32),
            pltpu.VMEM((B,), jnp.int32),
            pltpu.VMEM((B,), jnp.int32),
            pltpu.VMEM((B, 64), jnp.float32),
            pltpu.VMEM((B, D), jnp.float32),
            pltpu.VMEM((B, 64), jnp.float32),
            pltpu.VMEM((B, 16), jnp.float32),
        ]
        + [
            pltpu.VMEM_SHARED((N2, 64), jnp.float32),
            pltpu.VMEM_SHARED((N2, 16), jnp.float32),
        ]
        + 8 * [pltpu.SemaphoreType.DMA]
    ),
)


BN = 1000
_G10 = N // BN


def _half(w_ref, c):
    w = w_ref[...]
    return jnp.where(c == 1, w[:, 64:], w[:, :64])


def _kv_pack(z, k_ref, v_ref, c):
    kk = jnp.dot(z, _half(k_ref, c), preferred_element_type=jnp.float32)
    vv = jnp.dot(z, _half(v_ref, c), preferred_element_type=jnp.float32)
    return jnp.concatenate([kk, vv], axis=1)


def _tc_pre_body(x_ref, wp_ref, bp_ref, pos_ref, q_ref, k_ref, v_ref,
                 z_ref, qo_ref, kvo_ref):
    c = pl.program_id(0)
    z = jnp.dot(x_ref[...], wp_ref[...], preferred_element_type=jnp.float32)
    z = z + bp_ref[...] + pos_ref[...]
    z_ref[...] = z
    qo_ref[...] = jnp.dot(z, _half(q_ref, c),
                          preferred_element_type=jnp.float32)
    kvo_ref[...] = _kv_pack(z, k_ref, v_ref, c)


def _norm_res_ln(accs, accn, z, g, b):
    ssum = jnp.concatenate([accs[0], accs[1]], axis=1)
    n16 = jnp.concatenate([accn[0][:, :8], accn[1][:, :8]], axis=1)
    rowi = lax.broadcasted_iota(jnp.int32, (16, D), 0)
    hh = lax.broadcasted_iota(jnp.int32, (16, D), 1) // DH
    brd = (rowi == hh + (hh // 2) * 6).astype(jnp.float32)
    normw = jnp.dot(n16, brd, preferred_element_type=jnp.float32) + 1e-8
    res = ssum / normw + z
    mean = jnp.mean(res, axis=1, keepdims=True)
    cen = res - mean
    var = jnp.mean(cen * cen, axis=1, keepdims=True)
    return cen * lax.rsqrt(var + 1e-6) * g + b


def _tc_mid_body(accs_ref, accn_ref, z_ref, g_ref, b_ref,
                 q_ref, k_ref, v_ref, e_ref, qo_ref, kvo_ref):
    c = pl.program_id(0)
    e1 = _norm_res_ln(accs_ref[...], accn_ref[...],
                      z_ref[...], g_ref[...], b_ref[...])
    e_ref[...] = e1
    qo_ref[...] = jnp.dot(e1, _half(q_ref, c),
                          preferred_element_type=jnp.float32)
    kvo_ref[...] = _kv_pack(e1, k_ref, v_ref, c)


def _tc_post_body(accs_ref, accn_ref, z_ref, g_ref, b_ref,
                  w_ref, wb_ref, o_ref):
    e2 = _norm_res_ln(accs_ref[...], accn_ref[...],
                      z_ref[...], g_ref[...], b_ref[...])
    o_ref[...] = jnp.dot(e2, w_ref[...],
                         preferred_element_type=jnp.float32) + wb_ref[...]


def _row2(width):
    return pl.BlockSpec((BN, width), lambda c, i: (i, 0))


def _full2(shape):
    return pl.BlockSpec(shape, lambda c, i: tuple(0 for _ in shape))


def _acc2(width):
    return pl.BlockSpec((2, BN, width), lambda c, i: (0, i, 0))


_tc_pre = pl.pallas_call(
    _tc_pre_body,
    grid=(2, _G10),
    in_specs=[
        _row2(D), _full2((D, D)), _full2((1, D)), _full2((1, D)),
        _full2((D, D)), _full2((D, D)), _full2((D, D)),
    ],
    out_specs=[
        _row2(D),
        pl.BlockSpec((BN, 64), lambda c, i: (c * _G10 + i, 0)),
        pl.BlockSpec((BN, D), lambda c, i: (c * _G10 + i, 0)),
    ],
    out_shape=[
        jax.ShapeDtypeStruct((N, D), jnp.float32),
        jax.ShapeDtypeStruct((2 * N, 64), jnp.float32),
        jax.ShapeDtypeStruct((2 * N, D), jnp.float32),
    ],
)

_tc_mid = pl.pallas_call(
    _tc_mid_body,
    grid=(2, _G10),
    in_specs=[
        _acc2(64), _acc2(16),
        _row2(D), _full2((1, D)), _full2((1, D)),
        _full2((D, D)), _full2((D, D)), _full2((D, D)),
    ],
    out_specs=[
        _row2(D),
        pl.BlockSpec((BN, 64), lambda c, i: (c * _G10 + i, 0)),
        pl.BlockSpec((BN, D), lambda c, i: (c * _G10 + i, 0)),
    ],
    out_shape=[
        jax.ShapeDtypeStruct((N, D), jnp.float32),
        jax.ShapeDtypeStruct((2 * N, 64), jnp.float32),
        jax.ShapeDtypeStruct((2 * N, D), jnp.float32),
    ],
)

_tc_post = pl.pallas_call(
    _tc_post_body,
    grid=(_G10,),
    in_specs=[
        pl.BlockSpec((2, BN, 64), lambda i: (0, i, 0)),
        pl.BlockSpec((2, BN, 16), lambda i: (0, i, 0)),
        pl.BlockSpec((BN, D), lambda i: (i, 0)),
        pl.BlockSpec((1, D), lambda i: (0, 0)),
        pl.BlockSpec((1, D), lambda i: (0, 0)),
        pl.BlockSpec((D, D), lambda i: (0, 0)),
        pl.BlockSpec((1, D), lambda i: (0, 0)),
    ],
    out_specs=pl.BlockSpec((BN, D), lambda i: (i, 0)),
    out_shape=jax.ShapeDtypeStruct((N, D), jnp.float32),
)


@jax.jit
def kernel(node_features, edge_index, W_P_w, W_P_b, W_pos, q0, k0, v0, g0,
           b0, q1, k1, v1, g1, b1, inv_w, inv_b):
    rows = edge_index[0]
    cols = edge_index[1]
    z64 = jnp.zeros((B, 64), jnp.float32)
    z16 = jnp.zeros((B, 16), jnp.float32)

    z, q_a, kv_a = _tc_pre(node_features, W_P_w, W_P_b.reshape(1, D), W_pos,
                           q0, k0, v0)
    acc_s, acc_n = _sc_edge(q_a, kv_a, rows, cols, z64, z16)
    e1, q_b, kv_b = _tc_mid(acc_s, acc_n, z,
                            g0.reshape(1, D), b0.reshape(1, D), q1, k1, v1)
    acc_s, acc_n = _sc_edge(q_b, kv_b, rows, cols, z64, z16)
    return _tc_post(acc_s, acc_n, e1,
                    g1.reshape(1, D), b1.reshape(1, D), inv_w,
                    inv_b.reshape(1, D))


# speedup vs baseline: 1.0113x; 1.0113x over previous
import jax
import jax.numpy as jnp
from jax import lax
from jax.experimental import pallas as pl
from jax.experimental.pallas import tpu as pltpu
from jax.experimental.pallas import tpu_sc as plsc

N = 10000
E = 320000
D = 128
H = 4
DH = D // H

NC = 2
NS = 16
EPT = E // NS
B = 32
NBLK = EPT // B
N2 = 10240
RPT = N2 // NS


def _sc_edge_body(q_hbm, kv_hbm, rows_hbm, cols_hbm, z64_hbm, z16_hbm,
                  outs_hbm, outn_hbm,
                  rows_all, cols_all,
                  rows_va, rows2_va, cols2_va, qr_va, kvc_va, outs_va, outn_va,
                  rows_vb, rows2_vb, cols2_vb, qr_vb, kvc_vb, outs_vb, outn_vb,
                  acc_s, acc_n,
                  semqa, semka, semsa, semna, semqb, semkb, semsb, semnb):
    c = lax.axis_index("c")
    s = lax.axis_index("s")

    r0 = s * RPT
    pltpu.sync_copy(z64_hbm, outs_va)
    pltpu.sync_copy(z16_hbm, outn_va)
    pltpu.sync_copy(z16_hbm, outn_vb)
    for i in range(RPT // B):
        pltpu.sync_copy(outs_va, acc_s.at[pl.ds(r0 + i * B, B)])
        pltpu.sync_copy(outn_va, acc_n.at[pl.ds(r0 + i * B, B)])

    pltpu.sync_copy(rows_hbm.at[pl.ds(s * EPT, EPT)], rows_all)
    pltpu.sync_copy(cols_hbm.at[pl.ds(s * EPT, EPT)], cols_all)

    lanes = lax.iota(jnp.int32, 16)
    rowpat = lanes // 2
    colpat = lanes % 2
    coff = jnp.full((16,), c * N, jnp.int32)
    zv = jnp.zeros((16,), jnp.float32)

    plsc.subcore_barrier()

    def prep(bb, rows_v, rows2_v, cols2_v):
        for t in range(B // 16):
            rv = rows_all[pl.ds(bb * B + t * 16, 16)]
            cv = cols_all[pl.ds(bb * B + t * 16, 16)]
            rows_v[pl.ds(t * 16, 16)] = rv
            rows2_v[pl.ds(t * 16, 16)] = rv + coff
            cols2_v[pl.ds(t * 16, 16)] = cv + coff

    def fire(rows2_v, cols2_v, qr_v, kvc_v, semq, semk):
        pltpu.async_copy(q_hbm.at[rows2_v], qr_v, semq)
        pltpu.async_copy(kv_hbm.at[cols2_v], kvc_v, semk)

    def wait_gather(rows2_v, cols2_v, qr_v, kvc_v, semq, semk):
        pltpu.make_async_copy(q_hbm.at[rows2_v], qr_v, semq).wait()
        pltpu.make_async_copy(kv_hbm.at[cols2_v], kvc_v, semk).wait()

    def compute(qr_v, kvc_v, outs_v, outn_v):
        for g in range(B // 8):
            attvec = zv
            for j in range(16):
                e = 8 * g + j // 2
                hl = j % 2
                q0 = qr_v[e, pl.ds(hl * DH, 16)]
                q1 = qr_v[e, pl.ds(hl * DH + 16, 16)]
                k0 = kvc_v[e, pl.ds(hl * DH, 16)]
                k1 = kvc_v[e, pl.ds(hl * DH + 16, 16)]
                sv = jnp.full((16,), jnp.sum(q0 * k0 + q1 * k1))
                attvec = jnp.where(lanes == j, sv, attvec)
            attvec = jnp.minimum(jnp.maximum(attvec, -10.0), 10.0)
            ea = jnp.exp(attvec)
            plsc.store_scatter(outn_v, [rowpat + 8 * g, colpat], ea)
            for j in range(16):
                e = 8 * g + j // 2
                hl = j % 2
                av = jnp.full((16,), ea[j])
                v0 = kvc_v[e, pl.ds(64 + hl * DH, 16)]
                v1 = kvc_v[e, pl.ds(64 + hl * DH + 16, 16)]
                outs_v[e, pl.ds(hl * DH, 16)] = av * v0
                outs_v[e, pl.ds(hl * DH + 16, 16)] = av * v1

    def fire_scatter(outs_v, outn_v, rows_v, sems, semn):
        pltpu.async_copy(outs_v, acc_s.at[rows_v], sems, add=True)
        pltpu.async_copy(outn_v, acc_n.at[rows_v], semn, add=True)

    def wait_scatter(outs_v, outn_v, rows_v, sems, semn):
        pltpu.make_async_copy(outs_v, acc_s.at[rows_v], sems).wait()
        pltpu.make_async_copy(outn_v, acc_n.at[rows_v], semn).wait()

    prep(0, rows_va, rows2_va, cols2_va)
    fire(rows2_va, cols2_va, qr_va, kvc_va, semqa, semka)

    NPAIR = (NBLK - 1) // 2

    def pair(k, carry):
        bA = 2 * k
        @pl.when(k > 0)
        def _():
            wait_scatter(outs_vb, outn_vb, rows_vb, semsb, semnb)
        prep(bA + 1, rows_vb, rows2_vb, cols2_vb)
        fire(rows2_vb, cols2_vb, qr_vb, kvc_vb, semqb, semkb)
        wait_gather(rows2_va, cols2_va, qr_va, kvc_va, semqa, semka)
        compute(qr_va, kvc_va, outs_va, outn_va)
        fire_scatter(outs_va, outn_va, rows_va, semsa, semna)
        @pl.when(k < NPAIR - 1)
        def _():
            wait_scatter(outs_va, outn_va, rows_va, semsa, semna)
            prep(bA + 2, rows_va, rows2_va, cols2_va)
            fire(rows2_va, cols2_va, qr_va, kvc_va, semqa, semka)
        wait_gather(rows2_vb, cols2_vb, qr_vb, kvc_vb, semqb, semkb)
        compute(qr_vb, kvc_vb, outs_vb, outn_vb)
        fire_scatter(outs_vb, outn_vb, rows_vb, semsb, semnb)
        return carry

    lax.fori_loop(0, NPAIR, pair, 0)

    wait_scatter(outs_va, outn_va, rows_va, semsa, semna)
    wait_scatter(outs_vb, outn_vb, rows_vb, semsb, semnb)
    prep(NBLK - 1, rows_va, rows2_va, cols2_va)
    fire(rows2_va, cols2_va, qr_va, kvc_va, semqa, semka)
    wait_gather(rows2_va, cols2_va, qr_va, kvc_va, semqa, semka)
    compute(qr_va, kvc_va, outs_va, outn_va)
    fire_scatter(outs_va, outn_va, rows_va, semsa, semna)
    wait_scatter(outs_va, outn_va, rows_va, semsa, semna)

    plsc.subcore_barrier()

    for i in range(RPT // B):
        pltpu.sync_copy(acc_s.at[pl.ds(r0 + i * B, B)], outs_va)
        pltpu.sync_copy(outs_va, outs_hbm.at[c, pl.ds(r0 + i * B, B)])
        pltpu.sync_copy(acc_n.at[pl.ds(r0 + i * B, B)], outn_va)
        pltpu.sync_copy(outn_va, outn_hbm.at[c, pl.ds(r0 + i * B, B)])


_sc_edge = pl.kernel(
    _sc_edge_body,
    out_type=(
        jax.ShapeDtypeStruct((NC, N2, 64), jnp.float32),
        jax.ShapeDtypeStruct((NC, N2, 16), jnp.float32),
    ),
    mesh=plsc.VectorSubcoreMesh(core_axis_name="c", subcore_axis_name="s"),
    compiler_params=pltpu.CompilerParams(
        needs_layout_passes=False, use_tc_tiling_on_sc=False),
    scratch_types=(
        [
            pltpu.VMEM((EPT,), jnp.int32),
            pltpu.VMEM((EPT,), jnp.int32),
        ]
        + 2 * [
            pltpu.VMEM((B,), jnp.int32),
            pltpu.VMEM((B,), jnp.int32),
            pltpu.VMEM((B,), jnp.int32),
            pltpu.VMEM((B, 64), jnp.float32),
            pltpu.VMEM((B, D), jnp.float32),
            pltpu.VMEM((B, 64), jnp.float32),
            pltpu.VMEM((B, 16), jnp.float32),
        ]
        + [
            pltpu.VMEM_SHARED((N2, 64), jnp.float32),
            pltpu.VMEM_SHARED((N2, 16), jnp.float32),
        ]
        + 8 * [pltpu.SemaphoreType.DMA]
    ),
)


BN = 1000
_G10 = N // BN


def _half(w_ref, c):
    w = w_ref[...]
    return jnp.where(c == 1, w[:, 64:], w[:, :64])


def _kv_pack(z, k_ref, v_ref, c):
    kk = jnp.dot(z, _half(k_ref, c), preferred_element_type=jnp.float32)
    vv = jnp.dot(z, _half(v_ref, c), preferred_element_type=jnp.float32)
    return jnp.concatenate([kk, vv], axis=1)


def _tc_pre_body(x_ref, wp_ref, bp_ref, pos_ref, q_ref, k_ref, v_ref,
                 z_ref, qo_ref, kvo_ref):
    c = pl.program_id(0)
    z = jnp.dot(x_ref[...], wp_ref[...], preferred_element_type=jnp.float32)
    z = z + bp_ref[...] + pos_ref[...]
    z_ref[...] = z
    qo_ref[...] = jnp.dot(z, _half(q_ref, c),
                          preferred_element_type=jnp.float32)
    kvo_ref[...] = _kv_pack(z, k_ref, v_ref, c)


def _norm_res_ln(accs, accn, z, g, b):
    ssum = jnp.concatenate([accs[0], accs[1]], axis=1)
    n16 = jnp.concatenate([accn[0][:, :8], accn[1][:, :8]], axis=1)
    rowi = lax.broadcasted_iota(jnp.int32, (16, D), 0)
    hh = lax.broadcasted_iota(jnp.int32, (16, D), 1) // DH
    brd = (rowi == hh + (hh // 2) * 6).astype(jnp.float32)
    normw = jnp.dot(n16, brd, preferred_element_type=jnp.float32) + 1e-8
    res = ssum / normw + z
    mean = jnp.mean(res, axis=1, keepdims=True)
    cen = res - mean
    var = jnp.mean(cen * cen, axis=1, keepdims=True)
    return cen * lax.rsqrt(var + 1e-6) * g + b


def _tc_mid_body(accs_ref, accn_ref, z_ref, g_ref, b_ref,
                 q_ref, k_ref, v_ref, e_ref, qo_ref, kvo_ref):
    c = pl.program_id(0)
    e1 = _norm_res_ln(accs_ref[...], accn_ref[...],
                      z_ref[...], g_ref[...], b_ref[...])
    e_ref[...] = e1
    qo_ref[...] = jnp.dot(e1, _half(q_ref, c),
                          preferred_element_type=jnp.float32)
    kvo_ref[...] = _kv_pack(e1, k_ref, v_ref, c)


def _tc_post_body(accs_ref, accn_ref, z_ref, g_ref, b_ref,
                  w_ref, wb_ref, o_ref):
    e2 = _norm_res_ln(accs_ref[...], accn_ref[...],
                      z_ref[...], g_ref[...], b_ref[...])
    o_ref[...] = jnp.dot(e2, w_ref[...],
                         preferred_element_type=jnp.float32) + wb_ref[...]


def _row2(width):
    return pl.BlockSpec((BN, width), lambda c, i: (i, 0))


def _full2(shape):
    return pl.BlockSpec(shape, lambda c, i: tuple(0 for _ in shape))


def _acc2(width):
    return pl.BlockSpec((2, BN, width), lambda c, i: (0, i, 0))


_tc_pre = pl.pallas_call(
    _tc_pre_body,
    grid=(2, _G10),
    in_specs=[
        _row2(D), _full2((D, D)), _full2((1, D)), _full2((1, D)),
        _full2((D, D)), _full2((D, D)), _full2((D, D)),
    ],
    out_specs=[
        _row2(D),
        pl.BlockSpec((BN, 64), lambda c, i: (c * _G10 + i, 0)),
        pl.BlockSpec((BN, D), lambda c, i: (c * _G10 + i, 0)),
    ],
    out_shape=[
        jax.ShapeDtypeStruct((N, D), jnp.float32),
        jax.ShapeDtypeStruct((2 * N, 64), jnp.float32),
        jax.ShapeDtypeStruct((2 * N, D), jnp.float32),
    ],
)

_tc_mid = pl.pallas_call(
    _tc_mid_body,
    grid=(2, _G10),
    in_specs=[
        _acc2(64), _acc2(16),
        _row2(D), _full2((1, D)), _full2((1, D)),
        _full2((D, D)), _full2((D, D)), _full2((D, D)),
    ],
    out_specs=[
        _row2(D),
        pl.BlockSpec((BN, 64), lambda c, i: (c * _G10 + i, 0)),
        pl.BlockSpec((BN, D), lambda c, i: (c * _G10 + i, 0)),
    ],
    out_shape=[
        jax.ShapeDtypeStruct((N, D), jnp.float32),
        jax.ShapeDtypeStruct((2 * N, 64), jnp.float32),
        jax.ShapeDtypeStruct((2 * N, D), jnp.float32),
    ],
)

_tc_post = pl.pallas_call(
    _tc_post_body,
    grid=(_G10,),
    in_specs=[
        pl.BlockSpec((2, BN, 64), lambda i: (0, i, 0)),
        pl.BlockSpec((2, BN, 16), lambda i: (0, i, 0)),
        pl.BlockSpec((BN, D), lambda i: (i, 0)),
        pl.BlockSpec((1, D), lambda i: (0, 0)),
        pl.BlockSpec((1, D), lambda i: (0, 0)),
        pl.BlockSpec((D, D), lambda i: (0, 0)),
        pl.BlockSpec((1, D), lambda i: (0, 0)),
    ],
    out_specs=pl.BlockSpec((BN, D), lambda i: (i, 0)),
    out_shape=jax.ShapeDtypeStruct((N, D), jnp.float32),
)


@jax.jit
def kernel(node_features, edge_index, W_P_w, W_P_b, W_pos, q0, k0, v0, g0,
           b0, q1, k1, v1, g1, b1, inv_w, inv_b):
    rows = edge_index[0]
    cols = edge_index[1]
    z64 = jnp.zeros((B, 64), jnp.float32)
    z16 = jnp.zeros((B, 16), jnp.float32)

    z, q_a, kv_a = _tc_pre(node_features, W_P_w, W_P_b.reshape(1, D), W_pos,
                           q0, k0, v0)
    acc_s, acc_n = _sc_edge(q_a, kv_a, rows, cols, z64, z16)
    e1, q_b, kv_b = _tc_mid(acc_s, acc_n, z,
                            g0.reshape(1, D), b0.reshape(1, D), q1, k1, v1)
    acc_s, acc_n = _sc_edge(q_b, kv_b, rows, cols, z64, z16)
    return _tc_post(acc_s, acc_n, e1,
                    g1.reshape(1, D), b1.reshape(1, D), inv_w,
                    inv_b.reshape(1, D))
